# baseline (speedup 1.0000x reference)
import jax
import jax.numpy as jnp
from jax import lax
from jax.experimental import pallas as pl
from jax.experimental.pallas import tpu as pltpu
from jax.experimental.pallas import tpu_sc as plsc

_NC = 2
_NS = 16
_NW = _NC * _NS
_LANES = 16

_EMBED = 128
_BATCH = 16384
_B_PER_W = _BATCH // _NW
_CHUNK = 128
_NCHUNKS = _B_PER_W // _CHUNK
_NBUF = 2


def _sc_body(in_hbm, utab_hbm, itab_hbm, out_hbm,
             in_v, uidx_v, iidx_v,
             u0, u1, i0, i1, acc_v, out_v,
             sem_in, su0, su1, si0, si1):
    u_bufs = (u0, u1)
    i_bufs = (i0, i1)
    u_sems = (su0, su1)
    i_sems = (si0, si1)
    wid = lax.axis_index("s") * _NC + lax.axis_index("c")

    pltpu.async_copy(in_hbm.at[wid], in_v, sem_in).wait()

    lanes = lax.iota(jnp.int32, _LANES)
    lanes2 = lanes * 2
    zeros = jnp.zeros((_LANES,), jnp.int32)

    for c in range(_NCHUNKS):
        for g in range(_CHUNK // _LANES):
            off = 2 * (c * _CHUNK + g * _LANES)
            uidx_v[c, pl.ds(g * _LANES, _LANES)] = plsc.load_gather(
                in_v, [lanes2 + off])
            iidx_v[c, pl.ds(g * _LANES, _LANES)] = plsc.load_gather(
                in_v, [lanes2 + (off + 1)])

    def issue(c):
        s = c % _NBUF
        return (pltpu.async_copy(utab_hbm.at[uidx_v.at[c]], u_bufs[s], u_sems[s]),
                pltpu.async_copy(itab_hbm.at[iidx_v.at[c]], i_bufs[s], i_sems[s]))

    pending = [issue(c) for c in range(_NBUF)]
    for c in range(_NCHUNKS):
        cp_u, cp_i = pending[c]
        cp_u.wait()
        cp_i.wait()
        u_v = u_bufs[c % _NBUF]
        i_v = i_bufs[c % _NBUF]

        def group_body(g, _, c=c, u_v=u_v, i_v=i_v):
            base = g * _LANES
            for rr in range(_LANES):
                r = base + rr
                acc = u_v[r, pl.ds(0, _LANES)] * i_v[r, pl.ds(0, _LANES)]
                for j in range(1, _EMBED // _LANES):
                    acc = acc + (u_v[r, pl.ds(j * _LANES, _LANES)]
                                 * i_v[r, pl.ds(j * _LANES, _LANES)])
                acc_v[rr] = acc
            res = plsc.load_gather(acc_v, [lanes, zeros])
            for j in range(1, _LANES):
                res = res + plsc.load_gather(
                    acc_v, [lanes, jnp.full((_LANES,), j, jnp.int32)])
            out_v[pl.ds(c * _CHUNK + base, _LANES)] = res
            return 0

        lax.fori_loop(0, _CHUNK // _LANES, group_body, 0)
        if c + _NBUF < _NCHUNKS:
            pending.append(issue(c + _NBUF))

    pltpu.sync_copy(out_v, out_hbm.at[wid])


@jax.jit
def _cf_dot(inputs_flat, user_table, item_table):
    mesh = plsc.VectorSubcoreMesh(core_axis_name="c", subcore_axis_name="s",
                                  num_cores=_NC, num_subcores=_NS)
    k = pl.kernel(
        _sc_body,
        out_type=jax.ShapeDtypeStruct((_NW, _B_PER_W), jnp.float32),
        mesh=mesh,
        scratch_types=[
            pltpu.VMEM((2 * _B_PER_W,), jnp.int32),
            pltpu.VMEM((_NCHUNKS, _CHUNK), jnp.int32),
            pltpu.VMEM((_NCHUNKS, _CHUNK), jnp.int32),
            *[pltpu.VMEM((_CHUNK, _EMBED), jnp.float32) for _ in range(2 * _NBUF)],
            pltpu.VMEM((_LANES, _LANES), jnp.float32),
            pltpu.VMEM((_B_PER_W,), jnp.float32),
            *[pltpu.SemaphoreType.DMA for _ in range(1 + 2 * _NBUF)],
        ],
        compiler_params=pltpu.CompilerParams(needs_layout_passes=False),
    )
    return k(inputs_flat, user_table, item_table)


def kernel(inputs, user_table, item_table):
    idx = inputs.astype(jnp.int32).reshape(_NW, 2 * _B_PER_W)
    out = _cf_dot(idx, user_table, item_table)
    return out.reshape(_BATCH)


# speedup vs baseline: 1.2334x; 1.2334x over previous
import jax
import jax.numpy as jnp
from jax import lax
from jax.experimental import pallas as pl
from jax.experimental.pallas import tpu as pltpu
from jax.experimental.pallas import tpu_sc as plsc

_NC = 2
_NS = 16
_NW = _NC * _NS
_LANES = 16

_EMBED = 128
_BATCH = 16384
_B_PER_W = _BATCH // _NW
_CHUNK = 128
_NCHUNKS = _B_PER_W // _CHUNK
_NBUF = 2


def _sc_body(uidx_hbm, iidx_hbm, utab_hbm, itab_hbm, out_hbm,
             uidx_v, iidx_v, u0, u1, i0, i1, acc_v, out_v,
             sem_u, sem_i, su0, su1, si0, si1):
    u_bufs = (u0, u1)
    i_bufs = (i0, i1)
    u_sems = (su0, su1)
    i_sems = (si0, si1)
    wid = lax.axis_index("s") * _NC + lax.axis_index("c")

    cpu = pltpu.async_copy(uidx_hbm.at[wid], uidx_v, sem_u)
    cpi = pltpu.async_copy(iidx_hbm.at[wid], iidx_v, sem_i)
    cpu.wait()
    cpi.wait()

    lanes = lax.iota(jnp.int32, _LANES)
    zeros = jnp.zeros((_LANES,), jnp.int32)

    def issue(c):
        s = c % _NBUF
        return (pltpu.async_copy(utab_hbm.at[uidx_v.at[c]], u_bufs[s], u_sems[s]),
                pltpu.async_copy(itab_hbm.at[iidx_v.at[c]], i_bufs[s], i_sems[s]))

    pending = [issue(c) for c in range(_NBUF)]
    for c in range(_NCHUNKS):
        cp_u, cp_i = pending[c]
        cp_u.wait()
        cp_i.wait()
        u_v = u_bufs[c % _NBUF]
        i_v = i_bufs[c % _NBUF]

        def group_body(g, _, c=c, u_v=u_v, i_v=i_v):
            base = g * _LANES
            for rr in range(_LANES):
                r = base + rr
                acc = u_v[r, pl.ds(0, _LANES)] * i_v[r, pl.ds(0, _LANES)]
                for j in range(1, _EMBED // _LANES):
                    acc = acc + (u_v[r, pl.ds(j * _LANES, _LANES)]
                                 * i_v[r, pl.ds(j * _LANES, _LANES)])
                acc_v[rr] = acc
            res = plsc.load_gather(acc_v, [lanes, zeros])
            for j in range(1, _LANES):
                res = res + plsc.load_gather(
                    acc_v, [lanes, jnp.full((_LANES,), j, jnp.int32)])
            out_v[pl.ds(c * _CHUNK + base, _LANES)] = res
            return 0

        lax.fori_loop(0, _CHUNK // _LANES, group_body, 0)
        if c + _NBUF < _NCHUNKS:
            pending.append(issue(c + _NBUF))

    pltpu.sync_copy(out_v, out_hbm.at[wid])


@jax.jit
def _cf_dot(uidx, iidx, user_table, item_table):
    mesh = plsc.VectorSubcoreMesh(core_axis_name="c", subcore_axis_name="s",
                                  num_cores=_NC, num_subcores=_NS)
    k = pl.kernel(
        _sc_body,
        out_type=jax.ShapeDtypeStruct((_NW, _B_PER_W), jnp.float32),
        mesh=mesh,
        scratch_types=[
            pltpu.VMEM((_NCHUNKS, _CHUNK), jnp.int32),
            pltpu.VMEM((_NCHUNKS, _CHUNK), jnp.int32),
            *[pltpu.VMEM((_CHUNK, _EMBED), jnp.float32) for _ in range(2 * _NBUF)],
            pltpu.VMEM((_LANES, _LANES), jnp.float32),
            pltpu.VMEM((_B_PER_W,), jnp.float32),
            *[pltpu.SemaphoreType.DMA for _ in range(2 + 2 * _NBUF)],
        ],
        compiler_params=pltpu.CompilerParams(needs_layout_passes=False),
    )
    return k(uidx, iidx, user_table, item_table)


def kernel(inputs, user_table, item_table):
    idx = inputs.astype(jnp.int32)
    uidx = idx[:, 0].reshape(_NW, _NCHUNKS, _CHUNK)
    iidx = idx[:, 1].reshape(_NW, _NCHUNKS, _CHUNK)
    out = _cf_dot(uidx, iidx, user_table, item_table)
    return out.reshape(_BATCH)


# speedup vs baseline: 1.2839x; 1.0410x over previous
import jax
import jax.numpy as jnp
from jax import lax
from jax.experimental import pallas as pl
from jax.experimental.pallas import tpu as pltpu
from jax.experimental.pallas import tpu_sc as plsc

_NC = 2
_NS = 16
_NW = _NC * _NS
_LANES = 16

_EMBED = 128
_BATCH = 16384
_B_PER_W = _BATCH // _NW
_CHUNK = 128
_NCHUNKS = _B_PER_W // _CHUNK
_NBUF = 2


def _sc_body(uidx_hbm, iidx_hbm, utab_hbm, itab_hbm, out_hbm,
             uidx_v, iidx_v, u0, u1, i0, i1, acc_v, out_v,
             sem_u, sem_i, su0, su1, si0, si1):
    u_bufs = (u0, u1)
    i_bufs = (i0, i1)
    u_sems = (su0, su1)
    i_sems = (si0, si1)
    wid = lax.axis_index("s") * _NC + lax.axis_index("c")

    cpu = pltpu.async_copy(uidx_hbm.at[wid], uidx_v, sem_u)
    cpi = pltpu.async_copy(iidx_hbm.at[wid], iidx_v, sem_i)
    cpu.wait()
    cpi.wait()

    lanes = lax.iota(jnp.int32, _LANES)
    zeros = jnp.zeros((_LANES,), jnp.int32)

    def issue(c):
        s = c % _NBUF
        return (pltpu.async_copy(utab_hbm.at[uidx_v.at[c]], u_bufs[s], u_sems[s]),
                pltpu.async_copy(itab_hbm.at[iidx_v.at[c]], i_bufs[s], i_sems[s]))

    pending = [issue(c) for c in range(_NBUF)]
    for c in range(_NCHUNKS):
        cp_u, cp_i = pending[c]
        cp_u.wait()
        cp_i.wait()
        u_v = u_bufs[c % _NBUF]
        i_v = i_bufs[c % _NBUF]

        def group_body(g, _, c=c, u_v=u_v, i_v=i_v):
            base = g * _LANES
            for rr in range(_LANES):
                r = base + rr
                acc = u_v[r, pl.ds(0, _LANES)] * i_v[r, pl.ds(0, _LANES)]
                for j in range(1, _EMBED // _LANES):
                    acc = acc + (u_v[r, pl.ds(j * _LANES, _LANES)]
                                 * i_v[r, pl.ds(j * _LANES, _LANES)])
                acc_v[rr] = acc
            res = plsc.load_gather(acc_v, [lanes, zeros])
            for j in range(1, _LANES):
                res = res + plsc.load_gather(
                    acc_v, [lanes, jnp.full((_LANES,), j, jnp.int32)])
            out_v[c, pl.ds(base, _LANES)] = res
            return 0

        lax.fori_loop(0, _CHUNK // _LANES, group_body, 0)
        if c + _NBUF < _NCHUNKS:
            pending.append(issue(c + _NBUF))

    pltpu.sync_copy(out_v, out_hbm.at[wid])


@jax.jit
def _cf_dot(uidx, iidx, user_table, item_table):
    mesh = plsc.VectorSubcoreMesh(core_axis_name="c", subcore_axis_name="s",
                                  num_cores=_NC, num_subcores=_NS)
    k = pl.kernel(
        _sc_body,
        out_type=jax.ShapeDtypeStruct((_NW, _NCHUNKS, _CHUNK), jnp.float32),
        mesh=mesh,
        scratch_types=[
            pltpu.VMEM((_NCHUNKS, _CHUNK), jnp.int32),
            pltpu.VMEM((_NCHUNKS, _CHUNK), jnp.int32),
            *[pltpu.VMEM((_CHUNK, _EMBED), jnp.float32) for _ in range(2 * _NBUF)],
            pltpu.VMEM((_LANES, _LANES), jnp.float32),
            pltpu.VMEM((_NCHUNKS, _CHUNK), jnp.float32),
            *[pltpu.SemaphoreType.DMA for _ in range(2 + 2 * _NBUF)],
        ],
        compiler_params=pltpu.CompilerParams(needs_layout_passes=False),
    )
    return k(uidx, iidx, user_table, item_table)


def kernel(inputs, user_table, item_table):
    idx = inputs.astype(jnp.int32)
    uidx = idx[:, 0].reshape(_NW, _NCHUNKS, _CHUNK)
    iidx = idx[:, 1].reshape(_NW, _NCHUNKS, _CHUNK)
    out = _cf_dot(uidx, iidx, user_table, item_table)
    return out.reshape(_BATCH)


# speedup vs baseline: 1.2889x; 1.0039x over previous
import jax
import jax.numpy as jnp
from jax import lax
from jax.experimental import pallas as pl
from jax.experimental.pallas import tpu as pltpu
from jax.experimental.pallas import tpu_sc as plsc

_NC = 2
_NS = 16
_NW = _NC * _NS
_LANES = 16

_EMBED = 128
_BATCH = 16384
_B_PER_W = _BATCH // _NW
_CHUNK = 128
_NCHUNKS = _B_PER_W // _CHUNK
_NBUF = 2


def _sc_body(idx_hbm, utab_hbm, itab_hbm, out_hbm,
             idx_v, u0, u1, i0, i1, acc_v, out_v,
             sem_idx, sem_out, su0, su1, si0, si1):
    u_bufs = (u0, u1)
    i_bufs = (i0, i1)
    u_sems = (su0, su1)
    i_sems = (si0, si1)
    wid = lax.axis_index("s") * _NC + lax.axis_index("c")

    pltpu.async_copy(idx_hbm.at[wid], idx_v, sem_idx).wait()

    lanes = lax.iota(jnp.int32, _LANES)
    zeros = jnp.zeros((_LANES,), jnp.int32)

    def issue(c):
        s = c % _NBUF
        return (pltpu.async_copy(utab_hbm.at[idx_v.at[0, c]], u_bufs[s], u_sems[s]),
                pltpu.async_copy(itab_hbm.at[idx_v.at[1, c]], i_bufs[s], i_sems[s]))

    out_cps = []
    pending = [issue(c) for c in range(_NBUF)]
    for c in range(_NCHUNKS):
        cp_u, cp_i = pending[c]
        cp_u.wait()
        cp_i.wait()
        u_v = u_bufs[c % _NBUF]
        i_v = i_bufs[c % _NBUF]

        def group_body(g, _, c=c, u_v=u_v, i_v=i_v):
            base = g * _LANES
            for rr in range(_LANES):
                r = base + rr
                acc = u_v[r, pl.ds(0, _LANES)] * i_v[r, pl.ds(0, _LANES)]
                for j in range(1, _EMBED // _LANES):
                    acc = acc + (u_v[r, pl.ds(j * _LANES, _LANES)]
                                 * i_v[r, pl.ds(j * _LANES, _LANES)])
                acc_v[rr] = acc
            res = plsc.load_gather(acc_v, [lanes, zeros])
            for j in range(1, _LANES):
                res = res + plsc.load_gather(
                    acc_v, [lanes, jnp.full((_LANES,), j, jnp.int32)])
            out_v[c, pl.ds(base, _LANES)] = res
            return 0

        lax.fori_loop(0, _CHUNK // _LANES, group_body, 0)
        out_cps.append(
            pltpu.async_copy(out_v.at[c], out_hbm.at[wid, c], sem_out))
        if c + _NBUF < _NCHUNKS:
            pending.append(issue(c + _NBUF))

    for cp in out_cps:
        cp.wait()


@jax.jit
def _cf_dot(idx, user_table, item_table):
    mesh = plsc.VectorSubcoreMesh(core_axis_name="c", subcore_axis_name="s",
                                  num_cores=_NC, num_subcores=_NS)
    k = pl.kernel(
        _sc_body,
        out_type=jax.ShapeDtypeStruct((_NW, _NCHUNKS, _CHUNK), jnp.float32),
        mesh=mesh,
        scratch_types=[
            pltpu.VMEM((2, _NCHUNKS, _CHUNK), jnp.int32),
            *[pltpu.VMEM((_CHUNK, _EMBED), jnp.float32) for _ in range(2 * _NBUF)],
            pltpu.VMEM((_LANES, _LANES), jnp.float32),
            pltpu.VMEM((_NCHUNKS, _CHUNK), jnp.float32),
            *[pltpu.SemaphoreType.DMA for _ in range(2 + 2 * _NBUF)],
        ],
        compiler_params=pltpu.CompilerParams(needs_layout_passes=False),
    )
    return k(idx, user_table, item_table)


def kernel(inputs, user_table, item_table):
    idx = inputs.astype(jnp.int32)
    uidx = idx[:, 0].reshape(_NW, 1, _NCHUNKS, _CHUNK)
    iidx = idx[:, 1].reshape(_NW, 1, _NCHUNKS, _CHUNK)
    both = jnp.concatenate([uidx, iidx], axis=1)
    out = _cf_dot(both, user_table, item_table)
    return out.reshape(_BATCH)
